# baseline (speedup 1.0000x reference)
import functools

import jax
import jax.numpy as jnp
from jax import lax
from jax.experimental import pallas as pl
from jax.experimental.pallas import tpu as pltpu
from jax.experimental.pallas import tpu_sc as plsc

B = 16384
VOCAB = 100000
EMBED = 64
S = 64

NC, NS, L = 2, 16, 16
NW = NC * NS
BPW = B // NW
NG = (VOCAB + 127) // 128
VP = VOCAB // 2
HB = BPW // 2

_SC_PARAMS = pltpu.CompilerParams(needs_layout_passes=False,
                                  use_tc_tiling_on_sc=True,
                                  disable_bounds_checks=True)


def _sw_body(swf_hbm, swt_out, stack_v, out_v):
  wid = lax.axis_index("s") * NC + lax.axis_index("c")
  glo = wid * NG // NW
  ghi = (wid + 1) * NG // NW
  lane = lax.iota(jnp.int32, L)

  def grp(gi, _):
    pltpu.sync_copy(swf_hbm.at[pl.ds(0, EMBED), pl.ds(gi * 128, 128)],
                    stack_v)

    def rowa(a, _):
      for k in range(8):
        par = k // 4
        rows = lane + (k % 4) * L
        cols = jnp.zeros((L,), jnp.int32) + (2 * a + par)
        out_v[a, pl.ds(k * L, L)] = plsc.load_gather(stack_v, [rows, cols])
      return 0

    @pl.when(gi < NG - 1)
    def _():
      lax.fori_loop(0, 64, rowa, 0)
      pltpu.sync_copy(out_v, swt_out.at[pl.ds(gi * 64, 64)])

    @pl.when(gi == NG - 1)
    def _():
      lax.fori_loop(0, VP - 64 * (NG - 1), rowa, 0)
      pltpu.sync_copy(out_v.at[pl.ds(0, VP - 64 * (NG - 1))],
                      swt_out.at[pl.ds(64 * (NG - 1), VP - 64 * (NG - 1))])

    return 0

  lax.fori_loop(glo, ghi, grp, 0)


@jax.jit
def _sw_stage(swf):
  mesh = plsc.VectorSubcoreMesh(core_axis_name="c", subcore_axis_name="s",
                                num_cores=NC, num_subcores=NS)
  return pl.kernel(
      _sw_body,
      out_type=jax.ShapeDtypeStruct((VP, 128), jnp.float32),
      mesh=mesh,
      compiler_params=_SC_PARAMS,
      scratch_types=[
          pltpu.VMEM((EMBED, 128), jnp.float32),
          pltpu.VMEM((64, 128), jnp.float32),
      ],
  )(swf)


def _main_body(in_hbm, lab_hbm, sid_hbm, tab_hbm, swt_hbm, bias_hbm,
               emb_out, tl_out, wt_out, sb_out,
               in_v, lab_v, par_v, lpar_v, ridx_v, lridx_v,
               embg_v, labg_v, out_v, biasg_v, tl_v,
               ssid_v, srow_v, sg_v, wt_v, ssb_v,
               sem_e, sem_g, sem_b, sem_s1, sem_s2):
  wid = lax.axis_index("s") * NC + lax.axis_index("c")
  base = wid * BPW
  lane = lax.iota(jnp.int32, L)

  pltpu.sync_copy(in_hbm.at[pl.ds(base, BPW)], in_v)
  pltpu.sync_copy(lab_hbm.at[pl.ds(base, BPW)], lab_v)

  def prep(j, _):
    iv = in_v[pl.ds(j * L, L)]
    lv = lab_v[pl.ds(j * L, L)]
    par_v[pl.ds(j * L, L)] = iv & 1
    lpar_v[pl.ds(j * L, L)] = lv & 1
    return 0

  lax.fori_loop(0, BPW // L, prep, 0)

  b_cp = pltpu.async_copy(bias_hbm.at[lab_v], biasg_v, sem_b)

  @pl.when(wid == 0)
  def _():
    pltpu.sync_copy(sid_hbm, ssid_v)
    for j in range(S // L):
      srow_v[pl.ds(j * L, L)] = ssid_v[pl.ds(j * L, L)] >> 1
    pltpu.async_copy(swt_hbm.at[srow_v], sg_v, sem_s1).wait()
    pltpu.async_copy(bias_hbm.at[ssid_v], ssb_v, sem_s2).wait()

    for jc in range(S // L):
      sch = ssid_v[pl.ds(jc * L, L)]
      for i in range(L):
        s = jc * L + i
        sp = (sch[i] & 1) * 64
        rows = jnp.zeros((L,), jnp.int32) + s
        for k in range(EMBED // L):
          wt_v[pl.ds(s * EMBED + k * L, L)] = plsc.load_gather(
              sg_v, [rows, sp + lane + k * L])
    pltpu.sync_copy(wt_v, wt_out)
    pltpu.sync_copy(ssb_v, sb_out)

  b_cp.wait()

  for half in range(2):
    hb = half * HB

    def ridx(j, _):
      ridx_v[pl.ds(j * L, L)] = in_v[pl.ds(hb + j * L, L)] >> 1
      lridx_v[pl.ds(j * L, L)] = lab_v[pl.ds(hb + j * L, L)] >> 1
      return 0

    lax.fori_loop(0, HB // L, ridx, 0)
    e_cp = pltpu.async_copy(tab_hbm.at[ridx_v], embg_v, sem_e)
    g_cp = pltpu.async_copy(swt_hbm.at[lridx_v], labg_v, sem_g)
    e_cp.wait()
    g_cp.wait()

    def dot_j(j, _):
      rows = lane + j * L
      pe = par_v[pl.ds(hb + j * L, L)] * 64
      pg = lpar_v[pl.ds(hb + j * L, L)] * 64

      def dot_e(e, acc):
        ev = plsc.load_gather(embg_v, [rows, pe + e])
        gv = plsc.load_gather(labg_v, [rows, pg + e])
        return acc + ev * gv

      acc = lax.fori_loop(0, EMBED, dot_e, jnp.zeros((L,), jnp.float32))
      tl_v[pl.ds(hb + j * L, L)] = acc + biasg_v[pl.ds(hb + j * L, L)]
      return 0

    lax.fori_loop(0, HB // L, dot_j, 0)

    def asm(jm, _):
      pch = par_v[pl.ds(hb + jm * L, L)] * 64
      for i in range(L // 2):
        mm = jm * (L // 2) + i
        p0 = pch[2 * i]
        p1 = pch[2 * i + 1]
        r0 = jnp.zeros((L,), jnp.int32) + 2 * mm
        r1 = r0 + 1
        for k in range(EMBED // L):
          out_v[mm, pl.ds(k * L, L)] = plsc.load_gather(
              embg_v, [r0, p0 + lane + k * L])
          out_v[mm, pl.ds(EMBED + k * L, L)] = plsc.load_gather(
              embg_v, [r1, p1 + lane + k * L])
      return 0

    lax.fori_loop(0, HB // L, asm, 0)
    pltpu.sync_copy(out_v,
                    emb_out.at[pl.ds(wid * (BPW // 2) + half * (HB // 2),
                                     HB // 2)])

  pltpu.sync_copy(tl_v, tl_out.at[pl.ds(base, BPW)])


@jax.jit
def _main_stage(inputs, labels, sampled_ids, tab, swt, bias):
  mesh = plsc.VectorSubcoreMesh(core_axis_name="c", subcore_axis_name="s",
                                num_cores=NC, num_subcores=NS)
  return pl.kernel(
      _main_body,
      out_type=(
          jax.ShapeDtypeStruct((B // 2, 128), jnp.float32),
          jax.ShapeDtypeStruct((B,), jnp.float32),
          jax.ShapeDtypeStruct((S * EMBED,), jnp.float32),
          jax.ShapeDtypeStruct((S,), jnp.float32),
      ),
      mesh=mesh,
      compiler_params=_SC_PARAMS,
      scratch_types=[
          pltpu.VMEM((BPW,), jnp.int32),
          pltpu.VMEM((BPW,), jnp.int32),
          pltpu.VMEM((BPW,), jnp.int32),
          pltpu.VMEM((BPW,), jnp.int32),
          pltpu.VMEM((HB,), jnp.int32),
          pltpu.VMEM((HB,), jnp.int32),
          pltpu.VMEM((HB, 128), jnp.float32),
          pltpu.VMEM((HB, 128), jnp.float32),
          pltpu.VMEM((HB // 2, 128), jnp.float32),
          pltpu.VMEM((BPW,), jnp.float32),
          pltpu.VMEM((BPW,), jnp.float32),
          pltpu.VMEM((S,), jnp.int32),
          pltpu.VMEM((S,), jnp.int32),
          pltpu.VMEM((S, 128), jnp.float32),
          pltpu.VMEM((S * EMBED,), jnp.float32),
          pltpu.VMEM((S,), jnp.float32),
          pltpu.SemaphoreType.DMA,
          pltpu.SemaphoreType.DMA,
          pltpu.SemaphoreType.DMA,
          pltpu.SemaphoreType.DMA,
          pltpu.SemaphoreType.DMA,
      ],
  )(inputs, labels, sampled_ids, tab, swt, bias)


BLK = 4096


def _tc_body(emb_ref, w_ref, sb_ref, tl_ref, loss_ref):
  x = emb_ref[...]
  wt = w_ref[...]
  logits = lax.dot_general(x, wt, (((1,), (1,)), ((), ())),
                           preferred_element_type=jnp.float32) + sb_ref[...]
  neg = jnp.maximum(logits, 0.0) + jnp.log(1.0 + jnp.exp(-jnp.abs(logits)))
  t = tl_ref[0]
  pos = jnp.maximum(t, 0.0) - t + jnp.log(1.0 + jnp.exp(-jnp.abs(t)))
  loss_ref[0] = pos + jnp.sum(neg, axis=1)[None, :]


@jax.jit
def _tc_stage(emb, wt, sb, tl3d):
  return pl.pallas_call(
      _tc_body,
      grid=(B // BLK,),
      in_specs=[
          pl.BlockSpec((BLK, EMBED), lambda i: (i, 0)),
          pl.BlockSpec((S, EMBED), lambda i: (0, 0)),
          pl.BlockSpec((1, S), lambda i: (0, 0)),
          pl.BlockSpec((1, 1, BLK), lambda i: (i, 0, 0)),
      ],
      out_specs=pl.BlockSpec((1, 1, BLK), lambda i: (i, 0, 0)),
      out_shape=jax.ShapeDtypeStruct((B // BLK, 1, BLK), jnp.float32),
  )(emb, wt, sb, tl3d)


def kernel(inputs, target, sampled_ids, embedding_weights, score_weights,
           score_bias):
  inputs = inputs.astype(jnp.int32)
  labels = target[:, 0].astype(jnp.int32)
  sampled_ids = sampled_ids.astype(jnp.int32)
  tab = embedding_weights.reshape(B // 2, 128)
  swt = _sw_stage(score_weights)
  embp, tl, wtf, sbg = _main_stage(inputs, labels, sampled_ids, tab, swt,
                                   score_bias)
  emb = embp.reshape(B, EMBED)
  wt = wtf.reshape(S, EMBED)
  loss3d = _tc_stage(emb, wt, sbg.reshape(1, S), tl.reshape(B // BLK, 1, BLK))
  return emb, loss3d.reshape(B)


# speedup vs baseline: 1.0145x; 1.0145x over previous
import functools

import jax
import jax.numpy as jnp
from jax import lax
from jax.experimental import pallas as pl
from jax.experimental.pallas import tpu as pltpu
from jax.experimental.pallas import tpu_sc as plsc

B = 16384
VOCAB = 100000
EMBED = 64
S = 64

NC, NS, L = 2, 16, 16
NW = NC * NS
BPW = B // NW
NG = (VOCAB + 127) // 128
VP = VOCAB // 2
HB = BPW // 2

_SC_PARAMS = pltpu.CompilerParams(needs_layout_passes=False,
                                  use_tc_tiling_on_sc=True,
                                  disable_bounds_checks=True)


def _sw_body(swf_hbm, swt_out, stack_v, out_v):
  wid = lax.axis_index("s") * NC + lax.axis_index("c")
  glo = wid * NG // NW
  ghi = (wid + 1) * NG // NW
  lane = lax.iota(jnp.int32, L)

  def grp(gi, _):
    pltpu.sync_copy(swf_hbm.at[pl.ds(0, EMBED), pl.ds(gi * 128, 128)],
                    stack_v)

    def rowa(a, _):
      for k in range(8):
        par = k // 4
        rows = lane + (k % 4) * L
        cols = jnp.zeros((L,), jnp.int32) + (2 * a + par)
        out_v[a, pl.ds(k * L, L)] = plsc.load_gather(stack_v, [rows, cols])
      return 0

    @pl.when(gi < NG - 1)
    def _():
      lax.fori_loop(0, 64, rowa, 0, unroll=16)
      pltpu.sync_copy(out_v, swt_out.at[pl.ds(gi * 64, 64)])

    @pl.when(gi == NG - 1)
    def _():
      lax.fori_loop(0, VP - 64 * (NG - 1), rowa, 0, unroll=16)
      pltpu.sync_copy(out_v.at[pl.ds(0, VP - 64 * (NG - 1))],
                      swt_out.at[pl.ds(64 * (NG - 1), VP - 64 * (NG - 1))])

    return 0

  lax.fori_loop(glo, ghi, grp, 0)


@jax.jit
def _sw_stage(swf):
  mesh = plsc.VectorSubcoreMesh(core_axis_name="c", subcore_axis_name="s",
                                num_cores=NC, num_subcores=NS)
  return pl.kernel(
      _sw_body,
      out_type=jax.ShapeDtypeStruct((VP, 128), jnp.float32),
      mesh=mesh,
      compiler_params=_SC_PARAMS,
      scratch_types=[
          pltpu.VMEM((EMBED, 128), jnp.float32),
          pltpu.VMEM((64, 128), jnp.float32),
      ],
  )(swf)


def _main_body(in_hbm, lab_hbm, sid_hbm, tab_hbm, swt_hbm, bias_hbm,
               emb_out, tl_out, wt_out, sb_out,
               in_v, lab_v, par_v, lpar_v, ridx_v, lridx_v,
               embg_v, labg_v, out_v, biasg_v, tl_v,
               ssid_v, srow_v, sg_v, wt_v, ssb_v,
               sem_e, sem_g, sem_b, sem_s1, sem_s2):
  wid = lax.axis_index("s") * NC + lax.axis_index("c")
  base = wid * BPW
  lane = lax.iota(jnp.int32, L)

  pltpu.sync_copy(in_hbm.at[pl.ds(base, BPW)], in_v)
  pltpu.sync_copy(lab_hbm.at[pl.ds(base, BPW)], lab_v)

  def prep(j, _):
    iv = in_v[pl.ds(j * L, L)]
    lv = lab_v[pl.ds(j * L, L)]
    par_v[pl.ds(j * L, L)] = iv & 1
    lpar_v[pl.ds(j * L, L)] = lv & 1
    return 0

  lax.fori_loop(0, BPW // L, prep, 0, unroll=8)

  b_cp = pltpu.async_copy(bias_hbm.at[lab_v], biasg_v, sem_b)

  @pl.when(wid == 0)
  def _():
    pltpu.sync_copy(sid_hbm, ssid_v)
    for j in range(S // L):
      srow_v[pl.ds(j * L, L)] = ssid_v[pl.ds(j * L, L)] >> 1
    pltpu.async_copy(swt_hbm.at[srow_v], sg_v, sem_s1).wait()
    pltpu.async_copy(bias_hbm.at[ssid_v], ssb_v, sem_s2).wait()

    for jc in range(S // L):
      sch = ssid_v[pl.ds(jc * L, L)]
      for i in range(L):
        s = jc * L + i
        sp = (sch[i] & 1) * 64
        rows = jnp.zeros((L,), jnp.int32) + s
        for k in range(EMBED // L):
          wt_v[pl.ds(s * EMBED + k * L, L)] = plsc.load_gather(
              sg_v, [rows, sp + lane + k * L])
    pltpu.sync_copy(wt_v, wt_out)
    pltpu.sync_copy(ssb_v, sb_out)

  b_cp.wait()

  for half in range(2):
    hb = half * HB

    def ridx(j, _):
      ridx_v[pl.ds(j * L, L)] = in_v[pl.ds(hb + j * L, L)] >> 1
      lridx_v[pl.ds(j * L, L)] = lab_v[pl.ds(hb + j * L, L)] >> 1
      return 0

    lax.fori_loop(0, HB // L, ridx, 0, unroll=8)
    e_cp = pltpu.async_copy(tab_hbm.at[ridx_v], embg_v, sem_e)
    g_cp = pltpu.async_copy(swt_hbm.at[lridx_v], labg_v, sem_g)
    e_cp.wait()
    g_cp.wait()

    def dot_j(j, _):
      rows = lane + j * L
      pe = par_v[pl.ds(hb + j * L, L)] * 64
      pg = lpar_v[pl.ds(hb + j * L, L)] * 64

      def dot_e(e, acc):
        ev = plsc.load_gather(embg_v, [rows, pe + e])
        gv = plsc.load_gather(labg_v, [rows, pg + e])
        return acc + ev * gv

      acc = lax.fori_loop(0, EMBED, dot_e, jnp.zeros((L,), jnp.float32),
                          unroll=16)
      tl_v[pl.ds(hb + j * L, L)] = acc + biasg_v[pl.ds(hb + j * L, L)]
      return 0

    lax.fori_loop(0, HB // L, dot_j, 0)

    def asm(jm, _):
      pch = par_v[pl.ds(hb + jm * L, L)] * 64
      for i in range(L // 2):
        mm = jm * (L // 2) + i
        p0 = pch[2 * i]
        p1 = pch[2 * i + 1]
        r0 = jnp.zeros((L,), jnp.int32) + 2 * mm
        r1 = r0 + 1
        for k in range(EMBED // L):
          out_v[mm, pl.ds(k * L, L)] = plsc.load_gather(
              embg_v, [r0, p0 + lane + k * L])
          out_v[mm, pl.ds(EMBED + k * L, L)] = plsc.load_gather(
              embg_v, [r1, p1 + lane + k * L])
      return 0

    lax.fori_loop(0, HB // L, asm, 0, unroll=4)
    pltpu.sync_copy(out_v,
                    emb_out.at[pl.ds(wid * (BPW // 2) + half * (HB // 2),
                                     HB // 2)])

  pltpu.sync_copy(tl_v, tl_out.at[pl.ds(base, BPW)])


@jax.jit
def _main_stage(inputs, labels, sampled_ids, tab, swt, bias):
  mesh = plsc.VectorSubcoreMesh(core_axis_name="c", subcore_axis_name="s",
                                num_cores=NC, num_subcores=NS)
  return pl.kernel(
      _main_body,
      out_type=(
          jax.ShapeDtypeStruct((B // 2, 128), jnp.float32),
          jax.ShapeDtypeStruct((B,), jnp.float32),
          jax.ShapeDtypeStruct((S * EMBED,), jnp.float32),
          jax.ShapeDtypeStruct((S,), jnp.float32),
      ),
      mesh=mesh,
      compiler_params=_SC_PARAMS,
      scratch_types=[
          pltpu.VMEM((BPW,), jnp.int32),
          pltpu.VMEM((BPW,), jnp.int32),
          pltpu.VMEM((BPW,), jnp.int32),
          pltpu.VMEM((BPW,), jnp.int32),
          pltpu.VMEM((HB,), jnp.int32),
          pltpu.VMEM((HB,), jnp.int32),
          pltpu.VMEM((HB, 128), jnp.float32),
          pltpu.VMEM((HB, 128), jnp.float32),
          pltpu.VMEM((HB // 2, 128), jnp.float32),
          pltpu.VMEM((BPW,), jnp.float32),
          pltpu.VMEM((BPW,), jnp.float32),
          pltpu.VMEM((S,), jnp.int32),
          pltpu.VMEM((S,), jnp.int32),
          pltpu.VMEM((S, 128), jnp.float32),
          pltpu.VMEM((S * EMBED,), jnp.float32),
          pltpu.VMEM((S,), jnp.float32),
          pltpu.SemaphoreType.DMA,
          pltpu.SemaphoreType.DMA,
          pltpu.SemaphoreType.DMA,
          pltpu.SemaphoreType.DMA,
          pltpu.SemaphoreType.DMA,
      ],
  )(inputs, labels, sampled_ids, tab, swt, bias)


BLK = 8192


def _tc_body(emb_ref, w_ref, sb_ref, tl_ref, loss_ref):
  x = emb_ref[...]
  wt = w_ref[...]
  logits = lax.dot_general(x, wt, (((1,), (1,)), ((), ())),
                           preferred_element_type=jnp.float32) + sb_ref[...]
  neg = jnp.maximum(logits, 0.0) + jnp.log(1.0 + jnp.exp(-jnp.abs(logits)))
  t = tl_ref[0]
  pos = jnp.maximum(t, 0.0) - t + jnp.log(1.0 + jnp.exp(-jnp.abs(t)))
  loss_ref[0] = pos + jnp.sum(neg, axis=1)[None, :]


@jax.jit
def _tc_stage(emb, wt, sb, tl3d):
  return pl.pallas_call(
      _tc_body,
      grid=(B // BLK,),
      in_specs=[
          pl.BlockSpec((BLK, EMBED), lambda i: (i, 0)),
          pl.BlockSpec((S, EMBED), lambda i: (0, 0)),
          pl.BlockSpec((1, S), lambda i: (0, 0)),
          pl.BlockSpec((1, 1, BLK), lambda i: (i, 0, 0)),
      ],
      out_specs=pl.BlockSpec((1, 1, BLK), lambda i: (i, 0, 0)),
      out_shape=jax.ShapeDtypeStruct((B // BLK, 1, BLK), jnp.float32),
  )(emb, wt, sb, tl3d)


def kernel(inputs, target, sampled_ids, embedding_weights, score_weights,
           score_bias):
  inputs = inputs.astype(jnp.int32)
  labels = target[:, 0].astype(jnp.int32)
  sampled_ids = sampled_ids.astype(jnp.int32)
  tab = embedding_weights.reshape(B // 2, 128)
  swt = _sw_stage(score_weights)
  embp, tl, wtf, sbg = _main_stage(inputs, labels, sampled_ids, tab, swt,
                                   score_bias)
  emb = embp.reshape(B, EMBED)
  wt = wtf.reshape(S, EMBED)
  loss3d = _tc_stage(emb, wt, sbg.reshape(1, S), tl.reshape(B // BLK, 1, BLK))
  return emb, loss3d.reshape(B)


# speedup vs baseline: 1.1182x; 1.1022x over previous
import functools

import jax
import jax.numpy as jnp
from jax import lax
from jax.experimental import pallas as pl
from jax.experimental.pallas import tpu as pltpu
from jax.experimental.pallas import tpu_sc as plsc

B = 16384
VOCAB = 100000
EMBED = 64
S = 64

NC, NS, L = 2, 16, 16
NW = NC * NS
BPW = B // NW
NG = (VOCAB + 127) // 128
VP = VOCAB // 2
HB = BPW // 2

_SC_PARAMS = pltpu.CompilerParams(needs_layout_passes=False,
                                  use_tc_tiling_on_sc=True,
                                  disable_bounds_checks=True)


NBUF = 4


def _sw_body(swf_hbm, swt_out, st0, st1, st2, st3, ob0, ob1, ob2, ob3,
             is0, is1, is2, is3, os0, os1, os2, os3):
  sts = [st0, st1, st2, st3]
  obs = [ob0, ob1, ob2, ob3]
  isems = [is0, is1, is2, is3]
  osems = [os0, os1, os2, os3]
  wid = lax.axis_index("s") * NC + lax.axis_index("c")
  glo = wid * NG // NW
  ghi = (wid + 1) * NG // NW
  ngrp = ghi - glo
  lane = lax.iota(jnp.int32, L)

  for k in range(NBUF):
    pltpu.async_copy(
        swf_hbm.at[pl.ds(0, EMBED), pl.ds((glo + k) * 128, 128)],
        sts[k], isems[k])

  def step(k, n, gi):
    pltpu.make_async_copy(
        swf_hbm.at[pl.ds(0, EMBED), pl.ds(0, 128)], sts[k], isems[k]).wait()

    @pl.when(n >= NBUF)
    def _():
      pltpu.make_async_copy(obs[k], swt_out.at[pl.ds(0, 64)],
                            osems[k]).wait()

    def rowa(a, _):
      for q in range(8):
        par = q // 4
        rows = lane + (q % 4) * L
        cols = jnp.zeros((L,), jnp.int32) + (2 * a + par)
        obs[k][a, pl.ds(q * L, L)] = plsc.load_gather(sts[k], [rows, cols])
      return 0

    lax.fori_loop(0, 64, rowa, 0, unroll=16)

    @pl.when(n + NBUF < ngrp)
    def _():
      pltpu.async_copy(
          swf_hbm.at[pl.ds(0, EMBED), pl.ds((gi + NBUF) * 128, 128)],
          sts[k], isems[k])

    pltpu.async_copy(obs[k], swt_out.at[pl.ds(gi * 64, 64)], osems[k])

  def itr(n, _):
    gi = glo + n
    for k in range(NBUF):
      pl.when(n % NBUF == k)(lambda k=k: step(k, n, gi))
    return 0

  lax.fori_loop(0, ngrp, itr, 0)
  for k in range(NBUF):
    pltpu.make_async_copy(obs[k], swt_out.at[pl.ds(0, 64)], osems[k]).wait()


@jax.jit
def _sw_stage(swf):
  mesh = plsc.VectorSubcoreMesh(core_axis_name="c", subcore_axis_name="s",
                                num_cores=NC, num_subcores=NS)
  return pl.kernel(
      _sw_body,
      out_type=jax.ShapeDtypeStruct((64 * NG, 128), jnp.float32),
      mesh=mesh,
      compiler_params=_SC_PARAMS,
      scratch_types=(
          [pltpu.VMEM((EMBED, 128), jnp.float32)] * NBUF
          + [pltpu.VMEM((64, 128), jnp.float32)] * NBUF
          + [pltpu.SemaphoreType.DMA] * (2 * NBUF)
      ),
  )(swf)


def _main_body(in_hbm, lab_hbm, sid_hbm, tab_hbm, swt_hbm, bias_hbm,
               emb_out, tl_out, wt_out, sb_out,
               in_v, lab_v, par_v, lpar_v, ridx_v, lridx_v,
               embg_v, labg_v, out_v, biasg_v, tl_v,
               ssid_v, srow_v, sg_v, wt_v, ssb_v,
               sem_e, sem_g, sem_b, sem_s1, sem_s2):
  wid = lax.axis_index("s") * NC + lax.axis_index("c")
  base = wid * BPW
  lane = lax.iota(jnp.int32, L)

  pltpu.sync_copy(in_hbm.at[pl.ds(base, BPW)], in_v)
  pltpu.sync_copy(lab_hbm.at[pl.ds(base, BPW)], lab_v)

  def prep(j, _):
    iv = in_v[pl.ds(j * L, L)]
    lv = lab_v[pl.ds(j * L, L)]
    par_v[pl.ds(j * L, L)] = iv & 1
    lpar_v[pl.ds(j * L, L)] = lv & 1
    return 0

  lax.fori_loop(0, BPW // L, prep, 0, unroll=8)

  b_cp = pltpu.async_copy(bias_hbm.at[lab_v], biasg_v, sem_b)

  @pl.when(wid == 0)
  def _():
    pltpu.sync_copy(sid_hbm, ssid_v)
    for j in range(S // L):
      srow_v[pl.ds(j * L, L)] = ssid_v[pl.ds(j * L, L)] >> 1
    pltpu.async_copy(swt_hbm.at[srow_v], sg_v, sem_s1)
    pltpu.async_copy(bias_hbm.at[ssid_v], ssb_v, sem_s2)

  b_cp.wait()

  for half in range(2):
    hb = half * HB

    def ridx(j, _):
      ridx_v[pl.ds(j * L, L)] = in_v[pl.ds(hb + j * L, L)] >> 1
      lridx_v[pl.ds(j * L, L)] = lab_v[pl.ds(hb + j * L, L)] >> 1
      return 0

    lax.fori_loop(0, HB // L, ridx, 0, unroll=8)
    e_cp = pltpu.async_copy(tab_hbm.at[ridx_v], embg_v, sem_e)
    g_cp = pltpu.async_copy(swt_hbm.at[lridx_v], labg_v, sem_g)
    e_cp.wait()
    g_cp.wait()

    def dot_j(j, _):
      rows = lane + j * L
      pe = par_v[pl.ds(hb + j * L, L)] * 64
      pg = lpar_v[pl.ds(hb + j * L, L)] * 64

      def dot_e(e, acc):
        ev = plsc.load_gather(embg_v, [rows, pe + e])
        gv = plsc.load_gather(labg_v, [rows, pg + e])
        return acc + ev * gv

      acc = lax.fori_loop(0, EMBED, dot_e, jnp.zeros((L,), jnp.float32),
                          unroll=16)
      tl_v[pl.ds(hb + j * L, L)] = acc + biasg_v[pl.ds(hb + j * L, L)]
      return 0

    lax.fori_loop(0, HB // L, dot_j, 0)

    def asm(jm, _):
      pch = par_v[pl.ds(hb + jm * L, L)] * 64
      for i in range(L // 2):
        mm = jm * (L // 2) + i
        p0 = pch[2 * i]
        p1 = pch[2 * i + 1]
        r0 = jnp.zeros((L,), jnp.int32) + 2 * mm
        r1 = r0 + 1
        for k in range(EMBED // L):
          out_v[mm, pl.ds(k * L, L)] = plsc.load_gather(
              embg_v, [r0, p0 + lane + k * L])
          out_v[mm, pl.ds(EMBED + k * L, L)] = plsc.load_gather(
              embg_v, [r1, p1 + lane + k * L])
      return 0

    lax.fori_loop(0, HB // L, asm, 0, unroll=4)
    pltpu.sync_copy(out_v,
                    emb_out.at[pl.ds(wid * (BPW // 2) + half * (HB // 2),
                                     HB // 2)])

  pltpu.sync_copy(tl_v, tl_out.at[pl.ds(base, BPW)])

  @pl.when(wid == 0)
  def _():
    pltpu.make_async_copy(swt_hbm.at[srow_v], sg_v, sem_s1).wait()
    pltpu.make_async_copy(bias_hbm.at[ssid_v], ssb_v, sem_s2).wait()
    for jc in range(S // L):
      sch = ssid_v[pl.ds(jc * L, L)]
      for i in range(L):
        s = jc * L + i
        sp = (sch[i] & 1) * 64
        rows = jnp.zeros((L,), jnp.int32) + s
        for k in range(EMBED // L):
          wt_v[pl.ds(s * EMBED + k * L, L)] = plsc.load_gather(
              sg_v, [rows, sp + lane + k * L])
    pltpu.sync_copy(wt_v, wt_out)
    pltpu.sync_copy(ssb_v, sb_out)


@jax.jit
def _main_stage(inputs, labels, sampled_ids, tab, swt, bias):
  mesh = plsc.VectorSubcoreMesh(core_axis_name="c", subcore_axis_name="s",
                                num_cores=NC, num_subcores=NS)
  return pl.kernel(
      _main_body,
      out_type=(
          jax.ShapeDtypeStruct((B // 2, 128), jnp.float32),
          jax.ShapeDtypeStruct((B,), jnp.float32),
          jax.ShapeDtypeStruct((S * EMBED,), jnp.float32),
          jax.ShapeDtypeStruct((S,), jnp.float32),
      ),
      mesh=mesh,
      compiler_params=_SC_PARAMS,
      scratch_types=[
          pltpu.VMEM((BPW,), jnp.int32),
          pltpu.VMEM((BPW,), jnp.int32),
          pltpu.VMEM((BPW,), jnp.int32),
          pltpu.VMEM((BPW,), jnp.int32),
          pltpu.VMEM((HB,), jnp.int32),
          pltpu.VMEM((HB,), jnp.int32),
          pltpu.VMEM((HB, 128), jnp.float32),
          pltpu.VMEM((HB, 128), jnp.float32),
          pltpu.VMEM((HB // 2, 128), jnp.float32),
          pltpu.VMEM((BPW,), jnp.float32),
          pltpu.VMEM((BPW,), jnp.float32),
          pltpu.VMEM((S,), jnp.int32),
          pltpu.VMEM((S,), jnp.int32),
          pltpu.VMEM((S, 128), jnp.float32),
          pltpu.VMEM((S * EMBED,), jnp.float32),
          pltpu.VMEM((S,), jnp.float32),
          pltpu.SemaphoreType.DMA,
          pltpu.SemaphoreType.DMA,
          pltpu.SemaphoreType.DMA,
          pltpu.SemaphoreType.DMA,
          pltpu.SemaphoreType.DMA,
      ],
  )(inputs, labels, sampled_ids, tab, swt, bias)


BLK = 8192


def _tc_body(emb_ref, w_ref, sb_ref, tl_ref, loss_ref):
  x = emb_ref[...]
  wt = w_ref[...]
  logits = lax.dot_general(x, wt, (((1,), (1,)), ((), ())),
                           preferred_element_type=jnp.float32) + sb_ref[...]
  neg = jnp.maximum(logits, 0.0) + jnp.log(1.0 + jnp.exp(-jnp.abs(logits)))
  t = tl_ref[0]
  pos = jnp.maximum(t, 0.0) - t + jnp.log(1.0 + jnp.exp(-jnp.abs(t)))
  loss_ref[0] = pos + jnp.sum(neg, axis=1)[None, :]


@jax.jit
def _tc_stage(emb, wt, sb, tl3d):
  return pl.pallas_call(
      _tc_body,
      grid=(B // BLK,),
      in_specs=[
          pl.BlockSpec((BLK, EMBED), lambda i: (i, 0)),
          pl.BlockSpec((S, EMBED), lambda i: (0, 0)),
          pl.BlockSpec((1, S), lambda i: (0, 0)),
          pl.BlockSpec((1, 1, BLK), lambda i: (i, 0, 0)),
      ],
      out_specs=pl.BlockSpec((1, 1, BLK), lambda i: (i, 0, 0)),
      out_shape=jax.ShapeDtypeStruct((B // BLK, 1, BLK), jnp.float32),
  )(emb, wt, sb, tl3d)


def kernel(inputs, target, sampled_ids, embedding_weights, score_weights,
           score_bias):
  inputs = inputs.astype(jnp.int32)
  labels = target[:, 0].astype(jnp.int32)
  sampled_ids = sampled_ids.astype(jnp.int32)
  tab = embedding_weights.reshape(B // 2, 128)
  swt = _sw_stage(score_weights)
  embp, tl, wtf, sbg = _main_stage(inputs, labels, sampled_ids, tab, swt,
                                   score_bias)
  emb = embp.reshape(B, EMBED)
  wt = wtf.reshape(S, EMBED)
  loss3d = _tc_stage(emb, wt, sbg.reshape(1, S), tl.reshape(B // BLK, 1, BLK))
  return emb, loss3d.reshape(B)


# speedup vs baseline: 1.2323x; 1.1021x over previous
import functools

import jax
import jax.numpy as jnp
from jax import lax
from jax.experimental import pallas as pl
from jax.experimental.pallas import tpu as pltpu
from jax.experimental.pallas import tpu_sc as plsc

B = 16384
VOCAB = 100000
EMBED = 64
S = 64

NC, NS, L = 2, 16, 16
NW = NC * NS
BPW = B // NW
NG = (VOCAB + 127) // 128
VP = VOCAB // 2
HB = BPW // 2

_SC_PARAMS = pltpu.CompilerParams(needs_layout_passes=False,
                                  use_tc_tiling_on_sc=True,
                                  disable_bounds_checks=True)


NBUF = 4


def _sw_body(swf_hbm, swt_out, st0, st1, st2, st3, ob0, ob1, ob2, ob3,
             is0, is1, is2, is3, os0, os1, os2, os3):
  sts = [st0, st1, st2, st3]
  obs = [ob0, ob1, ob2, ob3]
  isems = [is0, is1, is2, is3]
  osems = [os0, os1, os2, os3]
  wid = lax.axis_index("s") * NC + lax.axis_index("c")
  glo = wid * NG // NW
  ghi = (wid + 1) * NG // NW
  ngrp = ghi - glo
  lane = lax.iota(jnp.int32, L)

  for k in range(NBUF):
    pltpu.async_copy(
        swf_hbm.at[pl.ds(0, EMBED), pl.ds((glo + k) * 128, 128)],
        sts[k].at[:, pl.ds(0, 128)], isems[k])

  def step(k, n, gi):
    pltpu.make_async_copy(
        swf_hbm.at[pl.ds(0, EMBED), pl.ds(0, 128)],
        sts[k].at[:, pl.ds(0, 128)], isems[k]).wait()

    @pl.when(n >= NBUF)
    def _():
      pltpu.make_async_copy(obs[k], swt_out.at[pl.ds(0, 64)],
                            osems[k]).wait()

    def rowa(a, _):
      for q in range(8):
        par = q // 4
        rows = lane + (q % 4) * L
        cols = jnp.zeros((L,), jnp.int32) + (2 * a + par)
        obs[k][a, pl.ds(q * L, L)] = plsc.load_gather(sts[k], [rows, cols])
      return 0

    lax.fori_loop(0, 64, rowa, 0, unroll=16)

    @pl.when(n + NBUF < ngrp)
    def _():
      pltpu.async_copy(
          swf_hbm.at[pl.ds(0, EMBED), pl.ds((gi + NBUF) * 128, 128)],
          sts[k].at[:, pl.ds(0, 128)], isems[k])

    pltpu.async_copy(obs[k], swt_out.at[pl.ds(gi * 64, 64)], osems[k])

  def itr(n, _):
    gi = glo + n
    for k in range(NBUF):
      pl.when(n % NBUF == k)(lambda k=k: step(k, n, gi))
    return 0

  lax.fori_loop(0, ngrp, itr, 0)
  for k in range(NBUF):
    pltpu.make_async_copy(obs[k], swt_out.at[pl.ds(0, 64)], osems[k]).wait()


@jax.jit
def _sw_stage(swf):
  mesh = plsc.VectorSubcoreMesh(core_axis_name="c", subcore_axis_name="s",
                                num_cores=NC, num_subcores=NS)
  return pl.kernel(
      _sw_body,
      out_type=jax.ShapeDtypeStruct((64 * NG, 128), jnp.float32),
      mesh=mesh,
      compiler_params=_SC_PARAMS,
      scratch_types=(
          [pltpu.VMEM((EMBED, 136), jnp.float32)] * NBUF
          + [pltpu.VMEM((64, 128), jnp.float32)] * NBUF
          + [pltpu.SemaphoreType.DMA] * (2 * NBUF)
      ),
  )(swf)


def _main_body(in_hbm, lab_hbm, sid_hbm, tab_hbm, swt_hbm, bias_hbm,
               emb_out, tl_out, wt_out, sb_out,
               in_v, lab_v, par_v, lpar_v, ridx_v, lridx_v,
               embg_v, labg_v, out_v, biasg_v, tl_v,
               ssid_v, srow_v, sg_v, wt_v, ssb_v,
               sem_e, sem_g, sem_b, sem_s1, sem_s2):
  wid = lax.axis_index("s") * NC + lax.axis_index("c")
  base = wid * BPW
  lane = lax.iota(jnp.int32, L)

  pltpu.sync_copy(in_hbm.at[pl.ds(base, BPW)], in_v)
  pltpu.sync_copy(lab_hbm.at[pl.ds(base, BPW)], lab_v)

  def prep(j, _):
    iv = in_v[pl.ds(j * L, L)]
    lv = lab_v[pl.ds(j * L, L)]
    par_v[pl.ds(j * L, L)] = iv & 1
    lpar_v[pl.ds(j * L, L)] = lv & 1
    return 0

  lax.fori_loop(0, BPW // L, prep, 0, unroll=8)

  b_cp = pltpu.async_copy(bias_hbm.at[lab_v], biasg_v, sem_b)

  @pl.when(wid == 0)
  def _():
    pltpu.sync_copy(sid_hbm, ssid_v)
    for j in range(S // L):
      srow_v[pl.ds(j * L, L)] = ssid_v[pl.ds(j * L, L)] >> 1
    pltpu.async_copy(swt_hbm.at[srow_v], sg_v, sem_s1)
    pltpu.async_copy(bias_hbm.at[ssid_v], ssb_v, sem_s2)

  b_cp.wait()

  for half in range(2):
    hb = half * HB

    def ridx(j, _):
      ridx_v[pl.ds(j * L, L)] = in_v[pl.ds(hb + j * L, L)] >> 1
      lridx_v[pl.ds(j * L, L)] = lab_v[pl.ds(hb + j * L, L)] >> 1
      return 0

    lax.fori_loop(0, HB // L, ridx, 0, unroll=8)
    e_cp = pltpu.async_copy(tab_hbm.at[ridx_v], embg_v, sem_e)
    g_cp = pltpu.async_copy(swt_hbm.at[lridx_v], labg_v, sem_g)
    e_cp.wait()
    g_cp.wait()

    def dot_j(j, _):
      pe = par_v[pl.ds(hb + j * L, L)] * 64
      pg = lpar_v[pl.ds(hb + j * L, L)] * 64
      out = jnp.zeros((L,), jnp.float32)
      for i in range(L):
        r = j * L + i
        se = pe[i]
        sg = pg[i]
        acc = jnp.zeros((L,), jnp.float32)
        for k in range(EMBED // L):
          acc = acc + (embg_v[r, pl.ds(se + k * L, L)]
                       * labg_v[r, pl.ds(sg + k * L, L)])
        out = jnp.where(lane == i, jnp.sum(acc), out)
      tl_v[pl.ds(hb + j * L, L)] = out + biasg_v[pl.ds(hb + j * L, L)]
      return 0

    lax.fori_loop(0, HB // L, dot_j, 0)

    def asm(jm, _):
      pch = par_v[pl.ds(hb + jm * L, L)] * 64
      for i in range(L // 2):
        mm = jm * (L // 2) + i
        p0 = pch[2 * i]
        p1 = pch[2 * i + 1]
        r0 = jnp.zeros((L,), jnp.int32) + 2 * mm
        r1 = r0 + 1
        for k in range(EMBED // L):
          out_v[mm, pl.ds(k * L, L)] = plsc.load_gather(
              embg_v, [r0, p0 + lane + k * L])
          out_v[mm, pl.ds(EMBED + k * L, L)] = plsc.load_gather(
              embg_v, [r1, p1 + lane + k * L])
      return 0

    lax.fori_loop(0, HB // L, asm, 0, unroll=4)
    pltpu.sync_copy(out_v,
                    emb_out.at[pl.ds(wid * (BPW // 2) + half * (HB // 2),
                                     HB // 2)])

  pltpu.sync_copy(tl_v, tl_out.at[pl.ds(base, BPW)])

  @pl.when(wid == 0)
  def _():
    pltpu.make_async_copy(swt_hbm.at[srow_v], sg_v, sem_s1).wait()
    pltpu.make_async_copy(bias_hbm.at[ssid_v], ssb_v, sem_s2).wait()
    for jc in range(S // L):
      sch = ssid_v[pl.ds(jc * L, L)]
      for i in range(L):
        s = jc * L + i
        sp = (sch[i] & 1) * 64
        rows = jnp.zeros((L,), jnp.int32) + s
        for k in range(EMBED // L):
          wt_v[pl.ds(s * EMBED + k * L, L)] = plsc.load_gather(
              sg_v, [rows, sp + lane + k * L])
    pltpu.sync_copy(wt_v, wt_out)
    pltpu.sync_copy(ssb_v, sb_out)


@jax.jit
def _main_stage(inputs, labels, sampled_ids, tab, swt, bias):
  mesh = plsc.VectorSubcoreMesh(core_axis_name="c", subcore_axis_name="s",
                                num_cores=NC, num_subcores=NS)
  return pl.kernel(
      _main_body,
      out_type=(
          jax.ShapeDtypeStruct((B // 2, 128), jnp.float32),
          jax.ShapeDtypeStruct((B,), jnp.float32),
          jax.ShapeDtypeStruct((S * EMBED,), jnp.float32),
          jax.ShapeDtypeStruct((S,), jnp.float32),
      ),
      mesh=mesh,
      compiler_params=_SC_PARAMS,
      scratch_types=[
          pltpu.VMEM((BPW,), jnp.int32),
          pltpu.VMEM((BPW,), jnp.int32),
          pltpu.VMEM((BPW,), jnp.int32),
          pltpu.VMEM((BPW,), jnp.int32),
          pltpu.VMEM((HB,), jnp.int32),
          pltpu.VMEM((HB,), jnp.int32),
          pltpu.VMEM((HB, 128), jnp.float32),
          pltpu.VMEM((HB, 128), jnp.float32),
          pltpu.VMEM((HB // 2, 128), jnp.float32),
          pltpu.VMEM((BPW,), jnp.float32),
          pltpu.VMEM((BPW,), jnp.float32),
          pltpu.VMEM((S,), jnp.int32),
          pltpu.VMEM((S,), jnp.int32),
          pltpu.VMEM((S, 128), jnp.float32),
          pltpu.VMEM((S * EMBED,), jnp.float32),
          pltpu.VMEM((S,), jnp.float32),
          pltpu.SemaphoreType.DMA,
          pltpu.SemaphoreType.DMA,
          pltpu.SemaphoreType.DMA,
          pltpu.SemaphoreType.DMA,
          pltpu.SemaphoreType.DMA,
      ],
  )(inputs, labels, sampled_ids, tab, swt, bias)


BLK = 8192


def _tc_body(emb_ref, w_ref, sb_ref, tl_ref, loss_ref):
  x = emb_ref[...]
  wt = w_ref[...]
  logits = lax.dot_general(x, wt, (((1,), (1,)), ((), ())),
                           preferred_element_type=jnp.float32) + sb_ref[...]
  neg = jnp.maximum(logits, 0.0) + jnp.log(1.0 + jnp.exp(-jnp.abs(logits)))
  t = tl_ref[0]
  pos = jnp.maximum(t, 0.0) - t + jnp.log(1.0 + jnp.exp(-jnp.abs(t)))
  loss_ref[0] = pos + jnp.sum(neg, axis=1)[None, :]


@jax.jit
def _tc_stage(emb, wt, sb, tl3d):
  return pl.pallas_call(
      _tc_body,
      grid=(B // BLK,),
      in_specs=[
          pl.BlockSpec((BLK, EMBED), lambda i: (i, 0)),
          pl.BlockSpec((S, EMBED), lambda i: (0, 0)),
          pl.BlockSpec((1, S), lambda i: (0, 0)),
          pl.BlockSpec((1, 1, BLK), lambda i: (i, 0, 0)),
      ],
      out_specs=pl.BlockSpec((1, 1, BLK), lambda i: (i, 0, 0)),
      out_shape=jax.ShapeDtypeStruct((B // BLK, 1, BLK), jnp.float32),
  )(emb, wt, sb, tl3d)


def kernel(inputs, target, sampled_ids, embedding_weights, score_weights,
           score_bias):
  inputs = inputs.astype(jnp.int32)
  labels = target[:, 0].astype(jnp.int32)
  sampled_ids = sampled_ids.astype(jnp.int32)
  tab = embedding_weights.reshape(B // 2, 128)
  swt = _sw_stage(score_weights)
  embp, tl, wtf, sbg = _main_stage(inputs, labels, sampled_ids, tab, swt,
                                   score_bias)
  emb = embp.reshape(B, EMBED)
  wt = wtf.reshape(S, EMBED)
  loss3d = _tc_stage(emb, wt, sbg.reshape(1, S), tl.reshape(B // BLK, 1, BLK))
  return emb, loss3d.reshape(B)


# speedup vs baseline: 2.4963x; 2.0257x over previous
import functools

import jax
import jax.numpy as jnp
from jax import lax
from jax.experimental import pallas as pl
from jax.experimental.pallas import tpu as pltpu
from jax.experimental.pallas import tpu_sc as plsc

B = 16384
VOCAB = 100000
EMBED = 64
S = 64

NC, NS, L = 2, 16, 16
NW = NC * NS
BPW = B // NW
VPAD = 100096
NCH = 17
CW = VPAD // NCH
EPW = EMBED // NW
NBUF = 4


def _flat_body(swf_hbm, flat_out, b0, b1, b2, b3, s0, s1, s2, s3):
  bufs = [b0, b1, b2, b3]
  sems = [s0, s1, s2, s3]
  wid = lax.axis_index("s") * NC + lax.axis_index("c")
  e0 = wid * EPW
  total = EPW * NCH

  def fire(n, k):
    e = e0 + n // NCH
    c = (n % NCH) * CW
    pltpu.async_copy(swf_hbm.at[e, pl.ds(c, CW)], bufs[k], sems[k])

  def drain_and_write(n, k):
    e = e0 + n // NCH
    c = (n % NCH) * CW
    pltpu.make_async_copy(swf_hbm.at[0, pl.ds(0, CW)], bufs[k],
                          sems[k]).wait()
    pltpu.sync_copy(bufs[k], flat_out.at[pl.ds(e * VPAD + c, CW)])

  for k in range(NBUF):
    fire(k, k)

  def step(n, k):
    drain_and_write(n, k)

    @pl.when(n + NBUF < total)
    def _():
      fire(n + NBUF, k)

  def itr(n, _):
    for k in range(NBUF):
      pl.when(n % NBUF == k)(functools.partial(step, n, k))
    return 0

  lax.fori_loop(0, total, itr, 0)


@jax.jit
def _flat_stage(swf):
  mesh = plsc.VectorSubcoreMesh(core_axis_name="c", subcore_axis_name="s",
                                num_cores=NC, num_subcores=NS)
  return pl.kernel(
      _flat_body,
      out_type=jax.ShapeDtypeStruct((EMBED * VPAD,), jnp.float32),
      mesh=mesh,
      compiler_params=pltpu.CompilerParams(needs_layout_passes=False,
                                           use_tc_tiling_on_sc=True,
                                           disable_bounds_checks=True),
      scratch_types=(
          [pltpu.VMEM((CW,), jnp.float32)] * NBUF
          + [pltpu.SemaphoreType.DMA] * NBUF
      ),
  )(swf)


def _sc_body(inputs_hbm, labels_hbm, sid_hbm, table_hbm, swf_hbm, bias_hbm,
             emb_out, tl_out, sw_out, sb_out,
             idx_v, lab_v, emb_v, gidx_v, g_v, biasg_v, tl_v,
             ssid_v, ssidx_v, ssw_v, ssb_v,
             sem_emb, sem_g, sem_b, sem_sw, sem_sb):
  wid = lax.axis_index("s") * NC + lax.axis_index("c")
  base = wid * BPW
  lane = lax.iota(jnp.int32, L)

  pltpu.sync_copy(inputs_hbm.at[pl.ds(base, BPW)], idx_v)
  pltpu.sync_copy(labels_hbm.at[pl.ds(base, BPW)], lab_v)

  emb_cp = pltpu.async_copy(table_hbm.at[idx_v], emb_v, sem_emb)

  e_off = [(lane + k * L) * VPAD for k in range(EMBED // L)]

  def build_j(j, _):
    lchunk = lab_v[pl.ds(j * L, L)]
    for i in range(L):
      lb = lchunk[i]
      b = j * L + i
      for k in range(EMBED // L):
        gidx_v[pl.ds(b * EMBED + k * L, L)] = e_off[k] + lb
    return 0

  lax.fori_loop(0, BPW // L, build_j, 0)

  emb_cp.wait()
  pltpu.sync_copy(emb_v, emb_out.at[pl.ds(base, BPW)])

  g_cp = pltpu.async_copy(swf_hbm.at[gidx_v], g_v, sem_g)
  b_cp = pltpu.async_copy(bias_hbm.at[lab_v], biasg_v, sem_b)

  pltpu.sync_copy(sid_hbm, ssid_v)
  for q in range(EMBED // NW):
    for j in range(S // L):
      ssidx_v[pl.ds(q * S + j * L, L)] = (
          ssid_v[pl.ds(j * L, L)] + (wid * (EMBED // NW) + q) * VPAD)
  sw_cp = pltpu.async_copy(swf_hbm.at[ssidx_v], ssw_v, sem_sw)

  @pl.when(wid == 0)
  def _():
    pltpu.async_copy(bias_hbm.at[ssid_v], ssb_v, sem_sb).wait()
    pltpu.sync_copy(ssb_v, sb_out)

  g_cp.wait()
  b_cp.wait()

  def dot_j(j, _):
    out = jnp.zeros((L,), jnp.float32)
    for i in range(L):
      b = j * L + i
      acc = jnp.zeros((L,), jnp.float32)
      for k in range(EMBED // L):
        acc = acc + (emb_v[b, pl.ds(k * L, L)]
                     * g_v[pl.ds(b * EMBED + k * L, L)])
      out = jnp.where(lane == i, jnp.sum(acc), out)
    tl_v[pl.ds(j * L, L)] = out + biasg_v[pl.ds(j * L, L)]
    return 0

  lax.fori_loop(0, BPW // L, dot_j, 0)
  pltpu.sync_copy(tl_v, tl_out.at[pl.ds(base, BPW)])

  sw_cp.wait()
  pltpu.sync_copy(ssw_v, sw_out.at[pl.ds(wid * (EMBED // NW) * S,
                                         (EMBED // NW) * S)])


@jax.jit
def _sc_stage(inputs, labels, sampled_ids, table, swf, bias):
  mesh = plsc.VectorSubcoreMesh(core_axis_name="c", subcore_axis_name="s",
                                num_cores=NC, num_subcores=NS)
  return pl.kernel(
      _sc_body,
      out_type=(
          jax.ShapeDtypeStruct((B, EMBED), jnp.float32),
          jax.ShapeDtypeStruct((B,), jnp.float32),
          jax.ShapeDtypeStruct((EMBED * S,), jnp.float32),
          jax.ShapeDtypeStruct((S,), jnp.float32),
      ),
      mesh=mesh,
      compiler_params=pltpu.CompilerParams(needs_layout_passes=False,
                                           use_tc_tiling_on_sc=False),
      scratch_types=[
          pltpu.VMEM((BPW,), jnp.int32),
          pltpu.VMEM((BPW,), jnp.int32),
          pltpu.VMEM((BPW, EMBED), jnp.float32),
          pltpu.VMEM((EMBED * BPW,), jnp.int32),
          pltpu.VMEM((EMBED * BPW,), jnp.float32),
          pltpu.VMEM((BPW,), jnp.float32),
          pltpu.VMEM((BPW,), jnp.float32),
          pltpu.VMEM((S,), jnp.int32),
          pltpu.VMEM((EMBED // NW * S,), jnp.int32),
          pltpu.VMEM((EMBED // NW * S,), jnp.float32),
          pltpu.VMEM((S,), jnp.float32),
          pltpu.SemaphoreType.DMA,
          pltpu.SemaphoreType.DMA,
          pltpu.SemaphoreType.DMA,
          pltpu.SemaphoreType.DMA,
          pltpu.SemaphoreType.DMA,
      ],
  )(inputs, labels, sampled_ids, table, swf, bias)


BLK = 8192


def _tc_body(emb_ref, w_ref, sb_ref, tl_ref, loss_ref):
  x = emb_ref[...]
  w = w_ref[...]
  logits = jnp.dot(x, w, preferred_element_type=jnp.float32) + sb_ref[...]
  neg = jnp.maximum(logits, 0.0) + jnp.log(1.0 + jnp.exp(-jnp.abs(logits)))
  t = tl_ref[0]
  pos = jnp.maximum(t, 0.0) - t + jnp.log(1.0 + jnp.exp(-jnp.abs(t)))
  loss_ref[0] = pos + jnp.sum(neg, axis=1)[None, :]


@jax.jit
def _tc_stage(emb, w, sb, tl3d):
  return pl.pallas_call(
      _tc_body,
      grid=(B // BLK,),
      in_specs=[
          pl.BlockSpec((BLK, EMBED), lambda i: (i, 0)),
          pl.BlockSpec((EMBED, S), lambda i: (0, 0)),
          pl.BlockSpec((1, S), lambda i: (0, 0)),
          pl.BlockSpec((1, 1, BLK), lambda i: (i, 0, 0)),
      ],
      out_specs=pl.BlockSpec((1, 1, BLK), lambda i: (i, 0, 0)),
      out_shape=jax.ShapeDtypeStruct((B // BLK, 1, BLK), jnp.float32),
  )(emb, w, sb, tl3d)


def kernel(inputs, target, sampled_ids, embedding_weights, score_weights,
           score_bias):
  inputs = inputs.astype(jnp.int32)
  labels = target[:, 0].astype(jnp.int32)
  sampled_ids = sampled_ids.astype(jnp.int32)
  swf = _flat_stage(score_weights)
  emb, tl, swg, sbg = _sc_stage(inputs, labels, sampled_ids,
                                embedding_weights, swf, score_bias)
  w = swg.reshape(EMBED, S)
  loss3d = _tc_stage(emb, w, sbg.reshape(1, S), tl.reshape(B // BLK, 1, BLK))
  return emb, loss3d.reshape(B)


# speedup vs baseline: 2.5288x; 1.0130x over previous
import functools

import jax
import jax.numpy as jnp
from jax import lax
from jax.experimental import pallas as pl
from jax.experimental.pallas import tpu as pltpu
from jax.experimental.pallas import tpu_sc as plsc

B = 16384
VOCAB = 100000
EMBED = 64
S = 64

NC, NS, L = 2, 16, 16
NW = NC * NS
BPW = B // NW
VPAD = 100096
NCH = 17
CW = VPAD // NCH
EPW = EMBED // NW
NBUF = 4


def _flat_body(swf_hbm, flat_out, b0, b1, b2, b3, s0, s1, s2, s3):
  bufs = [b0, b1, b2, b3]
  sems = [s0, s1, s2, s3]
  wid = lax.axis_index("s") * NC + lax.axis_index("c")
  e0 = wid * EPW
  total = EPW * NCH

  def fire(n, k):
    e = e0 + n // NCH
    c = (n % NCH) * CW
    pltpu.async_copy(swf_hbm.at[e, pl.ds(c, CW)], bufs[k], sems[k])

  def drain_and_write(n, k):
    e = e0 + n // NCH
    c = (n % NCH) * CW
    pltpu.make_async_copy(swf_hbm.at[0, pl.ds(0, CW)], bufs[k],
                          sems[k]).wait()
    pltpu.sync_copy(bufs[k], flat_out.at[pl.ds(e * VPAD + c, CW)])

  for k in range(NBUF):
    fire(k, k)

  def step(n, k):
    drain_and_write(n, k)

    @pl.when(n + NBUF < total)
    def _():
      fire(n + NBUF, k)

  def itr(n, _):
    for k in range(NBUF):
      pl.when(n % NBUF == k)(functools.partial(step, n, k))
    return 0

  lax.fori_loop(0, total, itr, 0)


@jax.jit
def _flat_stage(swf):
  mesh = plsc.VectorSubcoreMesh(core_axis_name="c", subcore_axis_name="s",
                                num_cores=NC, num_subcores=NS)
  return pl.kernel(
      _flat_body,
      out_type=jax.ShapeDtypeStruct((EMBED * VPAD,), jnp.float32),
      mesh=mesh,
      compiler_params=pltpu.CompilerParams(needs_layout_passes=False,
                                           use_tc_tiling_on_sc=True,
                                           disable_bounds_checks=True),
      scratch_types=(
          [pltpu.VMEM((CW,), jnp.float32)] * NBUF
          + [pltpu.SemaphoreType.DMA] * NBUF
      ),
  )(swf)


def _sc_body(inputs_hbm, labels_hbm, sid_hbm, table_hbm, swf_hbm, bias_hbm,
             emb_out, tl_out, sw_out, sb_out,
             idx_v, lab_v, emb_v, gidx_v, g_v, g2_v, biasg_v, tl_v,
             ssid_v, ssidx_v, ssw_v, ssb_v,
             sem_emb, sem_g, sem_b, sem_sw, sem_sb):
  wid = lax.axis_index("s") * NC + lax.axis_index("c")
  base = wid * BPW
  lane = lax.iota(jnp.int32, L)

  pltpu.sync_copy(inputs_hbm.at[pl.ds(base, BPW)], idx_v)
  pltpu.sync_copy(labels_hbm.at[pl.ds(base, BPW)], lab_v)

  emb_cp = pltpu.async_copy(table_hbm.at[idx_v], emb_v, sem_emb)

  e_off = [(lane + k * L) * VPAD for k in range(EMBED // L)]
  HBW = BPW // 2

  def build_j(hb, j, _):
    lchunk = lab_v[pl.ds(hb + j * L, L)]
    for i in range(L):
      lb = lchunk[i]
      b = j * L + i
      for k in range(EMBED // L):
        gidx_v[pl.ds(b * EMBED + k * L, L)] = e_off[k] + lb
    return 0

  lax.fori_loop(0, HBW // L, functools.partial(build_j, 0), 0)

  emb_cp.wait()
  pltpu.sync_copy(emb_v, emb_out.at[pl.ds(base, BPW)])

  g_cp = pltpu.async_copy(swf_hbm.at[gidx_v], g_v, sem_g)
  b_cp = pltpu.async_copy(bias_hbm.at[lab_v], biasg_v, sem_b)

  pltpu.sync_copy(sid_hbm, ssid_v)
  for q in range(EMBED // NW):
    for j in range(S // L):
      ssidx_v[pl.ds(q * S + j * L, L)] = (
          ssid_v[pl.ds(j * L, L)] + (wid * (EMBED // NW) + q) * VPAD)
  sw_cp = pltpu.async_copy(swf_hbm.at[ssidx_v], ssw_v, sem_sw)

  @pl.when(wid == 0)
  def _():
    pltpu.async_copy(bias_hbm.at[ssid_v], ssb_v, sem_sb).wait()
    pltpu.sync_copy(ssb_v, sb_out)

  g_cp.wait()
  b_cp.wait()

  lax.fori_loop(0, HBW // L, functools.partial(build_j, HBW), 0)
  g_cp2 = pltpu.async_copy(swf_hbm.at[gidx_v], g2_v, sem_g)

  def dot_j(hb, gv, j, _):
    out = jnp.zeros((L,), jnp.float32)
    for i in range(L):
      b = j * L + i
      acc = jnp.zeros((L,), jnp.float32)
      for k in range(EMBED // L):
        acc = acc + (emb_v[hb + b, pl.ds(k * L, L)]
                     * gv[pl.ds(b * EMBED + k * L, L)])
      out = jnp.where(lane == i, jnp.sum(acc), out)
    tl_v[pl.ds(hb + j * L, L)] = out + biasg_v[pl.ds(hb + j * L, L)]
    return 0

  lax.fori_loop(0, HBW // L, functools.partial(dot_j, 0, g_v), 0)
  g_cp2.wait()
  lax.fori_loop(0, HBW // L, functools.partial(dot_j, HBW, g2_v), 0)
  pltpu.sync_copy(tl_v, tl_out.at[pl.ds(base, BPW)])

  sw_cp.wait()
  pltpu.sync_copy(ssw_v, sw_out.at[pl.ds(wid * (EMBED // NW) * S,
                                         (EMBED // NW) * S)])


@jax.jit
def _sc_stage(inputs, labels, sampled_ids, table, swf, bias):
  mesh = plsc.VectorSubcoreMesh(core_axis_name="c", subcore_axis_name="s",
                                num_cores=NC, num_subcores=NS)
  return pl.kernel(
      _sc_body,
      out_type=(
          jax.ShapeDtypeStruct((B, 128), jnp.float32),
          jax.ShapeDtypeStruct((B,), jnp.float32),
          jax.ShapeDtypeStruct((EMBED * S,), jnp.float32),
          jax.ShapeDtypeStruct((S,), jnp.float32),
      ),
      mesh=mesh,
      compiler_params=pltpu.CompilerParams(needs_layout_passes=False,
                                           use_tc_tiling_on_sc=True,
                                           disable_bounds_checks=True),
      scratch_types=[
          pltpu.VMEM((BPW,), jnp.int32),
          pltpu.VMEM((BPW,), jnp.int32),
          pltpu.VMEM((BPW, 128), jnp.float32),
          pltpu.VMEM((EMBED * BPW // 2,), jnp.int32),
          pltpu.VMEM((EMBED * BPW // 2,), jnp.float32),
          pltpu.VMEM((EMBED * BPW // 2,), jnp.float32),
          pltpu.VMEM((BPW,), jnp.float32),
          pltpu.VMEM((BPW,), jnp.float32),
          pltpu.VMEM((S,), jnp.int32),
          pltpu.VMEM((EMBED // NW * S,), jnp.int32),
          pltpu.VMEM((EMBED // NW * S,), jnp.float32),
          pltpu.VMEM((S,), jnp.float32),
          pltpu.SemaphoreType.DMA,
          pltpu.SemaphoreType.DMA,
          pltpu.SemaphoreType.DMA,
          pltpu.SemaphoreType.DMA,
          pltpu.SemaphoreType.DMA,
      ],
  )(inputs, labels, sampled_ids, table, swf, bias)


BLK = 8192


def _tc_body(emb_ref, w_ref, sb_ref, tl_ref, loss_ref):
  x = emb_ref[...]
  w = w_ref[...]
  logits = jnp.dot(x, w, preferred_element_type=jnp.float32) + sb_ref[...]
  neg = jnp.maximum(logits, 0.0) + jnp.log(1.0 + jnp.exp(-jnp.abs(logits)))
  t = tl_ref[0]
  pos = jnp.maximum(t, 0.0) - t + jnp.log(1.0 + jnp.exp(-jnp.abs(t)))
  loss_ref[0] = pos + jnp.sum(neg, axis=1)[None, :]


@jax.jit
def _tc_stage(emb, w, sb, tl3d):
  return pl.pallas_call(
      _tc_body,
      grid=(B // BLK,),
      in_specs=[
          pl.BlockSpec((BLK, EMBED), lambda i: (i, 0)),
          pl.BlockSpec((EMBED, S), lambda i: (0, 0)),
          pl.BlockSpec((1, S), lambda i: (0, 0)),
          pl.BlockSpec((1, 1, BLK), lambda i: (i, 0, 0)),
      ],
      out_specs=pl.BlockSpec((1, 1, BLK), lambda i: (i, 0, 0)),
      out_shape=jax.ShapeDtypeStruct((B // BLK, 1, BLK), jnp.float32),
  )(emb, w, sb, tl3d)


def kernel(inputs, target, sampled_ids, embedding_weights, score_weights,
           score_bias):
  inputs = inputs.astype(jnp.int32)
  labels = target[:, 0].astype(jnp.int32)
  sampled_ids = sampled_ids.astype(jnp.int32)
  swf = _flat_stage(score_weights)
  tab = jnp.pad(embedding_weights, ((0, 0), (0, 128 - EMBED)))
  embp, tl, swg, sbg = _sc_stage(inputs, labels, sampled_ids,
                                 tab, swf, score_bias)
  emb = embp[:, :EMBED]
  w = swg.reshape(EMBED, S)
  loss3d = _tc_stage(emb, w, sbg.reshape(1, S), tl.reshape(B // BLK, 1, BLK))
  return emb, loss3d.reshape(B)
